# Initial kernel scaffold; baseline (speedup 1.0000x reference)
import jax
import jax.numpy as jnp
from jax.experimental import pallas as pl


def kernel(x, edge_index, edge_attr, W0, We0, as0, ad0, ae0, b0, lng0, lnb0, W1, We1, as1, ad1, ae1, b1, lng1, lnb1, W2, We2, as2, ad2, ae2, b2, lng2, lnb2, Wr1, br1, Wr2, br2, Wo1, bo1, Wo2, bo2, lngo, lnbo):
    raise NotImplementedError("write your pallas kernel here")


# speedup vs baseline: 12.4008x; 12.4008x over previous
import functools

import jax
import jax.numpy as jnp
from jax import lax
from jax.experimental import pallas as pl
from jax.experimental.pallas import tpu as pltpu
from jax.experimental.pallas import tpu_sc as plsc

N = 10000
E = 320000
H = 128
OUT = 256

NC = 2
NS = 16
NW = NC * NS
EPT = E // NW
CH = 128
NCHUNK = (EPT + CH - 1) // CH
EPT_PAD = NCHUNK * CH
RPT = N // NS

BLK = 80
NBLK = N // BLK

_f32 = jnp.float32
_sc_mesh = plsc.VectorSubcoreMesh(
    core_axis_name="c", subcore_axis_name="s", num_cores=NC, num_subcores=NS)


def _edge_body(h_hbm, s_hbm, d_hbm, srcp_hbm, dstp_hbm, aep_hbm,
               acc_hbm, den_hbm,
               s_v, d_v, src_c, dst_c, ae_c, coef_v, rows_v, den_v,
               out_sh, sem):
  cid = lax.axis_index("c")
  sid = lax.axis_index("s")
  wid = cid * NS + sid

  pltpu.sync_copy(s_hbm, s_v)
  pltpu.sync_copy(d_hbm, d_v)

  zero16 = jnp.zeros((16,), _f32)

  def _zden(i, carry):
    den_v[pl.ds(i * 16, 16)] = zero16
    return carry
  lax.fori_loop(0, N // 16, _zden, 0)

  def _zrows(i, carry):
    for j in range(H // 16):
      rows_v[i, pl.ds(j * 16, 16)] = zero16
    return carry
  lax.fori_loop(0, CH, _zrows, 0)

  zbase = sid * 632
  for t in range(4):
    pltpu.sync_copy(rows_v, out_sh.at[pl.ds(zbase + t * 128, 128)])

  @pl.when(sid < NS - 1)
  def _ztail():
    pltpu.sync_copy(rows_v.at[pl.ds(0, 120)],
                    out_sh.at[pl.ds(zbase + 512, 120)])

  @pl.when(sid == NS - 1)
  def _ztail2():
    pltpu.sync_copy(rows_v.at[pl.ds(0, 8)],
                    out_sh.at[pl.ds(zbase + 512, 8)])
  plsc.subcore_barrier()

  def _chunk(c, carry):
    pltpu.sync_copy(srcp_hbm.at[wid, c], src_c)
    pltpu.sync_copy(dstp_hbm.at[wid, c], dst_c)
    pltpu.sync_copy(aep_hbm.at[wid, c], ae_c)
    cp = pltpu.async_copy(h_hbm.at[src_c], rows_v, sem)
    for g in range(CH // 16):
      si = src_c[pl.ds(g * 16, 16)]
      di = dst_c[pl.ds(g * 16, 16)]
      ae = ae_c[pl.ds(g * 16, 16)]
      sv = plsc.load_gather(s_v, [si])
      dv = plsc.load_gather(d_v, [di])
      al = sv + dv + ae
      al = jnp.maximum(al, al * 0.2)
      ex = jnp.exp(al)
      valid = (c * CH + g * 16) < EPT
      ex = jnp.where(valid, ex, zero16)
      coef_v[pl.ds(g * 16, 16)] = ex
      plsc.addupdate_scatter(den_v, [di], ex)
    cp.wait()

    def _scale(r, carry2):
      cv = plsc.load_gather(coef_v, [jnp.full((16,), r, jnp.int32)])
      for j in range(H // 16):
        rows_v[r, pl.ds(j * 16, 16)] = rows_v[r, pl.ds(j * 16, 16)] * cv
      return carry2
    lax.fori_loop(0, CH, _scale, 0)

    pltpu.sync_copy(rows_v, out_sh.at[dst_c], add=True)
    return carry
  lax.fori_loop(0, NCHUNK, _chunk, 0)

  plsc.subcore_barrier()
  pltpu.sync_copy(den_v, den_hbm.at[wid])
  for t in range(4):
    pltpu.sync_copy(out_sh.at[pl.ds(zbase + t * 128, 128)],
                    acc_hbm.at[cid, pl.ds(zbase + t * 128, 128)])

  @pl.when(sid < NS - 1)
  def _dtail():
    pltpu.sync_copy(out_sh.at[pl.ds(zbase + 512, 120)],
                    acc_hbm.at[cid, pl.ds(zbase + 512, 120)])

  @pl.when(sid == NS - 1)
  def _dtail2():
    pltpu.sync_copy(out_sh.at[pl.ds(zbase + 512, 8)],
                    acc_hbm.at[cid, pl.ds(zbase + 512, 8)])


_edge_kernel = functools.partial(
    pl.kernel,
    out_type=(jax.ShapeDtypeStruct((NC, N, H), _f32),
              jax.ShapeDtypeStruct((NW, N), _f32)),
    mesh=_sc_mesh,
    compiler_params=pltpu.CompilerParams(needs_layout_passes=False),
    scratch_types=[
        pltpu.VMEM((N,), _f32),
        pltpu.VMEM((N,), _f32),
        pltpu.VMEM((CH,), jnp.int32),
        pltpu.VMEM((CH,), jnp.int32),
        pltpu.VMEM((CH,), _f32),
        pltpu.VMEM((CH,), _f32),
        pltpu.VMEM((CH, H), _f32),
        pltpu.VMEM((N,), _f32),
        pltpu.VMEM_SHARED((N, H), _f32),
        pltpu.SemaphoreType.DMA,
    ],
)(_edge_body)


def _deg_body(dstp_hbm, eatp_hbm, degp_hbm, lap_hbm,
              dst_c, eat_c, deg_v, la0, la1, la2, la3):
  cid = lax.axis_index("c")
  sid = lax.axis_index("s")
  wid = cid * NS + sid
  las = (la0, la1, la2, la3)

  zero16 = jnp.zeros((16,), _f32)
  one16 = jnp.ones((16,), _f32)

  def _z(i, carry):
    deg_v[pl.ds(i * 16, 16)] = zero16
    for j in range(4):
      las[j][pl.ds(i * 16, 16)] = zero16
    return carry
  lax.fori_loop(0, N // 16, _z, 0)

  def _chunk(c, carry):
    pltpu.sync_copy(dstp_hbm.at[wid, c], dst_c)
    pltpu.sync_copy(eatp_hbm.at[wid, c], eat_c)
    for g in range(CH // 16):
      di = dst_c[pl.ds(g * 16, 16)]
      valid = (c * CH + g * 16) < EPT
      ones_m = jnp.where(valid, one16, zero16)
      plsc.addupdate_scatter(deg_v, [di], ones_m)
      for j in range(4):
        ev = eat_c[pl.ds(j * CH + g * 16, 16)]
        ev = jnp.where(valid, ev, zero16)
        plsc.addupdate_scatter(las[j], [di], ev)
    return carry
  lax.fori_loop(0, NCHUNK, _chunk, 0)

  pltpu.sync_copy(deg_v, degp_hbm.at[wid])
  for j in range(4):
    pltpu.sync_copy(las[j], lap_hbm.at[j, wid])


_deg_kernel = functools.partial(
    pl.kernel,
    out_type=(jax.ShapeDtypeStruct((NW, N), _f32),
              jax.ShapeDtypeStruct((4, NW, N), _f32)),
    mesh=_sc_mesh,
    compiler_params=pltpu.CompilerParams(needs_layout_passes=False),
    scratch_types=[
        pltpu.VMEM((CH,), jnp.int32),
        pltpu.VMEM((4 * CH,), _f32),
        pltpu.VMEM((N,), _f32),
        pltpu.VMEM((N,), _f32),
        pltpu.VMEM((N,), _f32),
        pltpu.VMEM((N,), _f32),
        pltpu.VMEM((N,), _f32),
    ],
)(_deg_body)


def _ka_body(x_ref, w_ref, a2_ref, h_ref, sd_ref):
  h = jnp.dot(x_ref[...], w_ref[...], preferred_element_type=_f32)
  h_ref[...] = h
  sd_ref[...] = jnp.dot(h, a2_ref[...], preferred_element_type=_f32)


def _ka(x, w, a2):
  return pl.pallas_call(
      _ka_body,
      grid=(NBLK,),
      in_specs=[
          pl.BlockSpec((BLK, x.shape[1]), lambda i: (i, 0)),
          pl.BlockSpec(w.shape, lambda i: (0, 0)),
          pl.BlockSpec(a2.shape, lambda i: (0, 0)),
      ],
      out_specs=[
          pl.BlockSpec((BLK, H), lambda i: (i, 0)),
          pl.BlockSpec((BLK, 2), lambda i: (i, 0)),
      ],
      out_shape=(jax.ShapeDtypeStruct((N, H), _f32),
                 jax.ShapeDtypeStruct((N, 2), _f32)),
  )(x, w, a2)


def _mm_body(x_ref, m_ref, o_ref):
  o_ref[...] = jnp.dot(x_ref[...], m_ref[...], preferred_element_type=_f32)


def _alpha_e(ea_rs, m):
  return pl.pallas_call(
      _mm_body,
      grid=(NBLK,),
      in_specs=[
          pl.BlockSpec((BLK, 128), lambda i: (i, 0)),
          pl.BlockSpec(m.shape, lambda i: (0, 0)),
      ],
      out_specs=pl.BlockSpec((BLK, m.shape[1]), lambda i: (i, 0)),
      out_shape=jax.ShapeDtypeStruct((N, m.shape[1]), _f32),
  )(ea_rs, m)


def _red_body(degp_ref, lap_ref, sel_ref, ae3_ref, out_ref):
  deg = jnp.maximum(jnp.sum(degp_ref[...], axis=0), 1.0)
  la4 = lax.dot_general(sel_ref[...], lap_ref[...],
                        (((1,), (0,)), ((), ())),
                        preferred_element_type=_f32)
  la4 = la4 / deg[None, :]
  out_ref[...] = lax.dot_general(la4, ae3_ref[...],
                                 (((0,), (0,)), ((), ())),
                                 preferred_element_type=_f32)


def _reduce_la(degp, lap128, sel, ae3):
  return pl.pallas_call(
      _red_body,
      out_shape=jax.ShapeDtypeStruct((N, 3), _f32),
  )(degp, lap128, sel, ae3)


def _densum_body(denp_ref, out_ref):
  out_ref[...] = lax.dot_general(denp_ref[...], jnp.ones((NW, 1), _f32),
                                 (((0,), (0,)), ((), ())),
                                 preferred_element_type=_f32)


def _den_sum(denp):
  return pl.pallas_call(
      _densum_body,
      out_shape=jax.ShapeDtypeStruct((N, 1), _f32),
  )(denp)


def _gat_post(acc_ref, den_ref, sd_ref, lad_ref, h_ref, b_ref, g_ref,
              be_ref, layer):
  s = sd_ref[:, 0]
  dd = sd_ref[:, 1]
  asl = s + dd + lad_ref[:, layer]
  asl = jnp.maximum(asl, asl * 0.2)
  exs = jnp.exp(asl)
  h = h_ref[...]
  num = acc_ref[0] + acc_ref[1] + exs[:, None] * h
  den = den_ref[...] + exs[:, None]
  o = num / den + b_ref[...]
  m = jnp.mean(o, axis=1, keepdims=True)
  v = jnp.mean((o - m) ** 2, axis=1, keepdims=True)
  o = (o - m) * lax.rsqrt(v + 1e-5) * g_ref[...] + be_ref[...]
  return jnp.maximum(o, 0.0)


def _fin_mid_body(layer, acc_ref, den_ref, sd_ref, lad_ref, h_ref, b_ref,
                  g_ref, be_ref, w_ref, a2_ref, hn_ref, sdn_ref):
  o = _gat_post(acc_ref, den_ref, sd_ref, lad_ref, h_ref, b_ref, g_ref,
                be_ref, layer)
  hn = jnp.dot(o, w_ref[...], preferred_element_type=_f32)
  hn_ref[...] = hn
  sdn_ref[...] = jnp.dot(hn, a2_ref[...], preferred_element_type=_f32)


def _fin_last_body(acc_ref, den_ref, sd_ref, lad_ref, h_ref, b_ref,
                   g_ref, be_ref, hn_ref):
  hn_ref[...] = _gat_post(acc_ref, den_ref, sd_ref, lad_ref, h_ref, b_ref,
                          g_ref, be_ref, 2)


_PARAM_SPECS = [
    pl.BlockSpec((NC, BLK, H), lambda i: (0, i, 0)),
    pl.BlockSpec((BLK, 1), lambda i: (i, 0)),
    pl.BlockSpec((BLK, 2), lambda i: (i, 0)),
    pl.BlockSpec((BLK, 3), lambda i: (i, 0)),
    pl.BlockSpec((BLK, H), lambda i: (i, 0)),
    pl.BlockSpec((1, H), lambda i: (0, 0)),
    pl.BlockSpec((1, H), lambda i: (0, 0)),
    pl.BlockSpec((1, H), lambda i: (0, 0)),
]


def _finalize_mid(layer, acc, den, sd, lad, h, b, g, be, w, a2):
  return pl.pallas_call(
      functools.partial(_fin_mid_body, layer),
      grid=(NBLK,),
      in_specs=_PARAM_SPECS + [
          pl.BlockSpec((H, H), lambda i: (0, 0)),
          pl.BlockSpec((H, 2), lambda i: (0, 0)),
      ],
      out_specs=[
          pl.BlockSpec((BLK, H), lambda i: (i, 0)),
          pl.BlockSpec((BLK, 2), lambda i: (i, 0)),
      ],
      out_shape=(jax.ShapeDtypeStruct((N, H), _f32),
                 jax.ShapeDtypeStruct((N, 2), _f32)),
  )(acc, den, sd, lad, h, b, g, be, w, a2)


def _finalize_last(acc, den, sd, lad, h, b, g, be):
  return pl.pallas_call(
      _fin_last_body,
      grid=(NBLK,),
      in_specs=_PARAM_SPECS,
      out_specs=pl.BlockSpec((BLK, H), lambda i: (i, 0)),
      out_shape=jax.ShapeDtypeStruct((N, H), _f32),
  )(acc, den, sd, lad, h, b, g, be)


def _head_body(h_ref, wo1_ref, bo1_ref, wo2_ref, bo2_ref, g_ref, be_ref,
               o_ref, acc_ref):
  i = pl.program_id(0)

  @pl.when(i == 0)
  def _():
    acc_ref[...] = jnp.zeros_like(acc_ref)

  acc_ref[...] += jnp.sum(h_ref[...], axis=0, keepdims=True)

  @pl.when(i == NBLK - 1)
  def _():
    hg = acc_ref[...] * (1.0 / N)
    o1 = jnp.dot(hg, wo1_ref[...], preferred_element_type=_f32) + bo1_ref[...]
    o1 = jnp.maximum(o1, 0.0)
    o2 = jnp.dot(o1, wo2_ref[...], preferred_element_type=_f32) + bo2_ref[...]
    m = jnp.mean(o2, axis=1, keepdims=True)
    v = jnp.mean((o2 - m) ** 2, axis=1, keepdims=True)
    o_ref[...] = (o2 - m) * lax.rsqrt(v + 1e-5) * g_ref[...] + be_ref[...]


def _head(h3, wo1, bo1, wo2, bo2, g, be):
  return pl.pallas_call(
      _head_body,
      grid=(NBLK,),
      in_specs=[
          pl.BlockSpec((BLK, H), lambda i: (i, 0)),
          pl.BlockSpec((H, H), lambda i: (0, 0)),
          pl.BlockSpec((1, H), lambda i: (0, 0)),
          pl.BlockSpec((H, OUT), lambda i: (0, 0)),
          pl.BlockSpec((1, OUT), lambda i: (0, 0)),
          pl.BlockSpec((1, OUT), lambda i: (0, 0)),
          pl.BlockSpec((1, OUT), lambda i: (0, 0)),
      ],
      out_specs=pl.BlockSpec((1, OUT), lambda i: (0, 0)),
      out_shape=jax.ShapeDtypeStruct((1, OUT), _f32),
      scratch_shapes=[pltpu.VMEM((1, H), _f32)],
  )(h3, wo1, bo1, wo2, bo2, g, be)


def _pad_tiles(arr):
  a = arr.reshape(NW, EPT)
  a = jnp.pad(a, ((0, 0), (0, EPT_PAD - EPT)))
  return a.reshape(NW, NCHUNK, CH)


def kernel(x, edge_index, edge_attr, W0, We0, as0, ad0, ae0, b0, lng0, lnb0,
           W1, We1, as1, ad1, ae1, b1, lng1, lnb1,
           W2, We2, as2, ad2, ae2, b2, lng2, lnb2,
           Wr1, br1, Wr2, br2, Wo1, bo1, Wo2, bo2, lngo, lnbo):
  src = edge_index[0].astype(jnp.int32)
  dst = edge_index[1].astype(jnp.int32)
  srcp = _pad_tiles(src)
  dstp = _pad_tiles(dst)

  ae3 = jnp.stack([We0 @ ae0, We1 @ ae1, We2 @ ae2], axis=1)
  m = jnp.kron(jnp.eye(32, dtype=_f32), ae3)
  ea_rs = edge_attr.reshape(N, 128)
  al_all = _alpha_e(ea_rs, m).reshape(N, 32, 3).reshape(E, 3)
  aeps = [_pad_tiles(al_all[:, l]) for l in range(3)]

  eatp = jnp.pad(edge_attr.T.reshape(4, NW, EPT),
                 ((0, 0), (0, 0), (0, EPT_PAD - EPT))).reshape(
                     4, NW, NCHUNK, CH).transpose(1, 2, 0, 3).reshape(
                     NW, NCHUNK, 4 * CH)
  degp, lap = _deg_kernel(dstp, eatp)
  sel = jnp.kron(jnp.eye(4, dtype=_f32), jnp.ones((1, NW), _f32))
  lad = _reduce_la(degp, lap.reshape(4 * NW, N), sel, ae3)

  params = [
      (W0, as0, ad0, b0, lng0, lnb0),
      (W1, as1, ad1, b1, lng1, lnb1),
      (W2, as2, ad2, b2, lng2, lnb2),
  ]
  a2s = [jnp.stack([p[1], p[2]], axis=1) for p in params]

  h, sd = _ka(x, W0, a2s[0])
  for l in range(3):
    acc, denp = _edge_kernel(h, sd[:, 0], sd[:, 1], srcp, dstp, aeps[l])
    den = _den_sum(denp)
    W, _, _, b, g, be = params[l]
    if l < 2:
      Wn = params[l + 1][0]
      h, sd = _finalize_mid(l, acc, den, sd, lad,
                            h, b.reshape(1, H), g.reshape(1, H),
                            be.reshape(1, H), Wn, a2s[l + 1])
    else:
      h3 = _finalize_last(acc, den, sd, lad, h, b.reshape(1, H),
                          g.reshape(1, H), be.reshape(1, H))

  return _head(h3, Wo1, bo1.reshape(1, H), Wo2, bo2.reshape(1, OUT),
               lngo.reshape(1, OUT), lnbo.reshape(1, OUT))


# speedup vs baseline: 13.0548x; 1.0527x over previous
import functools

import jax
import jax.numpy as jnp
from jax import lax
from jax.experimental import pallas as pl
from jax.experimental.pallas import tpu as pltpu
from jax.experimental.pallas import tpu_sc as plsc

N = 10000
E = 320000
H = 128
OUT = 256

NC = 2
NS = 16
NW = NC * NS
EPT = E // NW
CH = 64
NCHUNK = 158
EPT_PAD = NCHUNK * CH
NBODY = NCHUNK // 2
RPT = N // NS

BLK = 80
NBLK = N // BLK

_f32 = jnp.float32
_sc_mesh = plsc.VectorSubcoreMesh(
    core_axis_name="c", subcore_axis_name="s", num_cores=NC, num_subcores=NS)


def _edge_body(h_hbm, s_hbm, d_hbm, srcp_hbm, dstp_hbm, aep_hbm,
               acc_hbm, den_hbm,
               s_v, d_v, src0, src1, dst0, dst1, ae0, ae1, coef_v,
               rows0, rows1, den_v, out_sh,
               sem_i0, sem_i1, sem_g0, sem_g1, sem_s0, sem_s1):
  cid = lax.axis_index("c")
  sid = lax.axis_index("s")
  wid = cid * NS + sid

  pltpu.sync_copy(s_hbm, s_v)
  pltpu.sync_copy(d_hbm, d_v)

  zero16 = jnp.zeros((16,), _f32)
  zi16 = jnp.zeros((16,), jnp.int32)
  NG = CH // 16

  def _zden(i, carry):
    den_v[pl.ds(i * 16, 16)] = zero16
    return carry
  lax.fori_loop(0, N // 16, _zden, 0)

  def _zrows(i, carry):
    for j in range(H // 16):
      rows0[i, pl.ds(j * 16, 16)] = zero16
    return carry
  lax.fori_loop(0, CH, _zrows, 0)

  zbase = sid * 632
  for t in range(8):
    pltpu.sync_copy(rows0, out_sh.at[pl.ds(zbase + t * 64, 64)])

  @pl.when(sid < NS - 1)
  def _ztail():
    pltpu.sync_copy(rows0, out_sh.at[pl.ds(zbase + 512, 64)])
    pltpu.sync_copy(rows0.at[pl.ds(0, 56)],
                    out_sh.at[pl.ds(zbase + 576, 56)])

  @pl.when(sid == NS - 1)
  def _ztail2():
    pltpu.sync_copy(rows0.at[pl.ds(0, 8)],
                    out_sh.at[pl.ds(zbase + 512, 8)])
  plsc.subcore_barrier()

  def _issue_idx(c, srcb, dstb, aeb, semi):
    pltpu.async_copy(srcp_hbm.at[wid, c], srcb, semi)
    pltpu.async_copy(dstp_hbm.at[wid, c], dstb, semi)
    pltpu.async_copy(aep_hbm.at[wid, c], aeb, semi)

  def _wait_idx(c, srcb, dstb, aeb, semi):
    pltpu.make_async_copy(srcp_hbm.at[wid, c], srcb, semi).wait()
    pltpu.make_async_copy(dstp_hbm.at[wid, c], dstb, semi).wait()
    pltpu.make_async_copy(aep_hbm.at[wid, c], aeb, semi).wait()

  def _issue_gather(srcb, rowsb, semg):
    pltpu.async_copy(h_hbm.at[srcb], rowsb, semg)

  def _wait_gather(srcb, rowsb, semg):
    pltpu.make_async_copy(h_hbm.at[srcb], rowsb, semg).wait()

  def _issue_scatter(dstb, rowsb, sems):
    pltpu.async_copy(rowsb, out_sh.at[dstb], sems, add=True)

  def _wait_scatter(dstb, rowsb, sems):
    pltpu.make_async_copy(rowsb, out_sh.at[dstb], sems).wait()

  def _process(c, srcb, dstb, aeb, rowsb):
    for g in range(NG):
      si = srcb[pl.ds(g * 16, 16)]
      di = dstb[pl.ds(g * 16, 16)]
      ae = aeb[pl.ds(g * 16, 16)]
      sv = plsc.load_gather(s_v, [si])
      dv = plsc.load_gather(d_v, [di])
      al = sv + dv + ae
      al = jnp.maximum(al, al * 0.2)
      ex = jnp.exp(al)
      valid = (c * CH + g * 16) < EPT
      ex = jnp.where(valid, ex, zero16)
      coef_v[pl.ds(g * 16, 16)] = ex
      plsc.addupdate_scatter(den_v, [di], ex)
    def _scale(r, carry2):
      cv = plsc.load_gather(coef_v, [jnp.full((16,), r, jnp.int32)])
      for j in range(H // 16):
        rowsb[r, pl.ds(j * 16, 16)] = rowsb[r, pl.ds(j * 16, 16)] * cv
      return carry2
    lax.fori_loop(0, CH, _scale, 0)

  _issue_idx(0, src0, dst0, ae0, sem_i0)
  _issue_idx(1, src1, dst1, ae1, sem_i1)
  _wait_idx(0, src0, dst0, ae0, sem_i0)
  _issue_gather(src0, rows0, sem_g0)

  def _body(i, carry):
    c0 = 2 * i
    c1 = 2 * i + 1
    _wait_gather(src0, rows0, sem_g0)

    @pl.when(i > 0)
    def _():
      _wait_scatter(dst1, rows1, sem_s1)
      _issue_idx(c1, src1, dst1, ae1, sem_i1)

    _wait_idx(c1, src1, dst1, ae1, sem_i1)
    _issue_gather(src1, rows1, sem_g1)
    _process(c0, src0, dst0, ae0, rows0)
    _issue_scatter(dst0, rows0, sem_s0)

    _wait_gather(src1, rows1, sem_g1)
    _process(c1, src1, dst1, ae1, rows1)
    _issue_scatter(dst1, rows1, sem_s1)

    @pl.when(i < NBODY - 1)
    def _():
      _wait_scatter(dst0, rows0, sem_s0)
      _issue_idx(c0 + 2, src0, dst0, ae0, sem_i0)
      _wait_idx(c0 + 2, src0, dst0, ae0, sem_i0)
      _issue_gather(src0, rows0, sem_g0)
    return carry
  lax.fori_loop(0, NBODY, _body, 0)

  _wait_scatter(dst0, rows0, sem_s0)
  _wait_scatter(dst1, rows1, sem_s1)

  plsc.subcore_barrier()
  pltpu.sync_copy(den_v, den_hbm.at[wid])
  for t in range(8):
    pltpu.sync_copy(out_sh.at[pl.ds(zbase + t * 64, 64)],
                    acc_hbm.at[cid, pl.ds(zbase + t * 64, 64)])

  @pl.when(sid < NS - 1)
  def _dtail():
    pltpu.sync_copy(out_sh.at[pl.ds(zbase + 512, 120)],
                    acc_hbm.at[cid, pl.ds(zbase + 512, 120)])

  @pl.when(sid == NS - 1)
  def _dtail2():
    pltpu.sync_copy(out_sh.at[pl.ds(zbase + 512, 8)],
                    acc_hbm.at[cid, pl.ds(zbase + 512, 8)])


_edge_kernel = functools.partial(
    pl.kernel,
    out_type=(jax.ShapeDtypeStruct((NC, N, H), _f32),
              jax.ShapeDtypeStruct((NW, N), _f32)),
    mesh=_sc_mesh,
    compiler_params=pltpu.CompilerParams(needs_layout_passes=False),
    scratch_types=[
        pltpu.VMEM((N,), _f32),
        pltpu.VMEM((N,), _f32),
        pltpu.VMEM((CH,), jnp.int32),
        pltpu.VMEM((CH,), jnp.int32),
        pltpu.VMEM((CH,), jnp.int32),
        pltpu.VMEM((CH,), jnp.int32),
        pltpu.VMEM((CH,), _f32),
        pltpu.VMEM((CH,), _f32),
        pltpu.VMEM((CH,), _f32),
        pltpu.VMEM((CH, H), _f32),
        pltpu.VMEM((CH, H), _f32),
        pltpu.VMEM((N,), _f32),
        pltpu.VMEM_SHARED((N, H), _f32),
        pltpu.SemaphoreType.DMA,
        pltpu.SemaphoreType.DMA,
        pltpu.SemaphoreType.DMA,
        pltpu.SemaphoreType.DMA,
        pltpu.SemaphoreType.DMA,
        pltpu.SemaphoreType.DMA,
    ],
)(_edge_body)


def _deg_body(dstp_hbm, eatp_hbm, degp_hbm, lap_hbm,
              dst_c, eat_c, deg_v, la0, la1, la2, la3):
  cid = lax.axis_index("c")
  sid = lax.axis_index("s")
  wid = cid * NS + sid
  las = (la0, la1, la2, la3)

  zero16 = jnp.zeros((16,), _f32)
  one16 = jnp.ones((16,), _f32)

  def _z(i, carry):
    deg_v[pl.ds(i * 16, 16)] = zero16
    for j in range(4):
      las[j][pl.ds(i * 16, 16)] = zero16
    return carry
  lax.fori_loop(0, N // 16, _z, 0)

  def _chunk(c, carry):
    pltpu.sync_copy(dstp_hbm.at[wid, c], dst_c)
    pltpu.sync_copy(eatp_hbm.at[wid, c], eat_c)
    for g in range(CH // 16):
      di = dst_c[pl.ds(g * 16, 16)]
      valid = (c * CH + g * 16) < EPT
      ones_m = jnp.where(valid, one16, zero16)
      plsc.addupdate_scatter(deg_v, [di], ones_m)
      for j in range(4):
        ev = eat_c[pl.ds(j * CH + g * 16, 16)]
        ev = jnp.where(valid, ev, zero16)
        plsc.addupdate_scatter(las[j], [di], ev)
    return carry
  lax.fori_loop(0, NCHUNK, _chunk, 0)

  pltpu.sync_copy(deg_v, degp_hbm.at[wid])
  for j in range(4):
    pltpu.sync_copy(las[j], lap_hbm.at[j, wid])


_deg_kernel = functools.partial(
    pl.kernel,
    out_type=(jax.ShapeDtypeStruct((NW, N), _f32),
              jax.ShapeDtypeStruct((4, NW, N), _f32)),
    mesh=_sc_mesh,
    compiler_params=pltpu.CompilerParams(needs_layout_passes=False),
    scratch_types=[
        pltpu.VMEM((CH,), jnp.int32),
        pltpu.VMEM((4 * CH,), _f32),
        pltpu.VMEM((N,), _f32),
        pltpu.VMEM((N,), _f32),
        pltpu.VMEM((N,), _f32),
        pltpu.VMEM((N,), _f32),
        pltpu.VMEM((N,), _f32),
    ],
)(_deg_body)


def _ka_body(x_ref, w_ref, a2_ref, h_ref, sd_ref):
  h = jnp.dot(x_ref[...], w_ref[...], preferred_element_type=_f32)
  h_ref[...] = h
  sd_ref[...] = jnp.dot(h, a2_ref[...], preferred_element_type=_f32)


def _ka(x, w, a2):
  return pl.pallas_call(
      _ka_body,
      grid=(NBLK,),
      in_specs=[
          pl.BlockSpec((BLK, x.shape[1]), lambda i: (i, 0)),
          pl.BlockSpec(w.shape, lambda i: (0, 0)),
          pl.BlockSpec(a2.shape, lambda i: (0, 0)),
      ],
      out_specs=[
          pl.BlockSpec((BLK, H), lambda i: (i, 0)),
          pl.BlockSpec((BLK, 2), lambda i: (i, 0)),
      ],
      out_shape=(jax.ShapeDtypeStruct((N, H), _f32),
                 jax.ShapeDtypeStruct((N, 2), _f32)),
  )(x, w, a2)


def _mm_body(x_ref, m_ref, o_ref):
  o_ref[...] = jnp.dot(x_ref[...], m_ref[...], preferred_element_type=_f32)


def _alpha_e(ea_rs, m):
  return pl.pallas_call(
      _mm_body,
      grid=(NBLK,),
      in_specs=[
          pl.BlockSpec((BLK, 128), lambda i: (i, 0)),
          pl.BlockSpec(m.shape, lambda i: (0, 0)),
      ],
      out_specs=pl.BlockSpec((BLK, m.shape[1]), lambda i: (i, 0)),
      out_shape=jax.ShapeDtypeStruct((N, m.shape[1]), _f32),
  )(ea_rs, m)


def _red_body(degp_ref, lap_ref, sel_ref, ae3_ref, out_ref):
  deg = jnp.maximum(jnp.sum(degp_ref[...], axis=0), 1.0)
  la4 = lax.dot_general(sel_ref[...], lap_ref[...],
                        (((1,), (0,)), ((), ())),
                        preferred_element_type=_f32)
  la4 = la4 / deg[None, :]
  out_ref[...] = lax.dot_general(la4, ae3_ref[...],
                                 (((0,), (0,)), ((), ())),
                                 preferred_element_type=_f32)


def _reduce_la(degp, lap128, sel, ae3):
  return pl.pallas_call(
      _red_body,
      out_shape=jax.ShapeDtypeStruct((N, 3), _f32),
  )(degp, lap128, sel, ae3)


def _densum_body(denp_ref, out_ref):
  out_ref[...] = lax.dot_general(denp_ref[...], jnp.ones((NW, 1), _f32),
                                 (((0,), (0,)), ((), ())),
                                 preferred_element_type=_f32)


def _den_sum(denp):
  return pl.pallas_call(
      _densum_body,
      out_shape=jax.ShapeDtypeStruct((N, 1), _f32),
  )(denp)


def _gat_post(acc_ref, den_ref, sd_ref, lad_ref, h_ref, b_ref, g_ref,
              be_ref, layer):
  s = sd_ref[:, 0]
  dd = sd_ref[:, 1]
  asl = s + dd + lad_ref[:, layer]
  asl = jnp.maximum(asl, asl * 0.2)
  exs = jnp.exp(asl)
  h = h_ref[...]
  num = acc_ref[0] + acc_ref[1] + exs[:, None] * h
  den = den_ref[...] + exs[:, None]
  o = num / den + b_ref[...]
  m = jnp.mean(o, axis=1, keepdims=True)
  v = jnp.mean((o - m) ** 2, axis=1, keepdims=True)
  o = (o - m) * lax.rsqrt(v + 1e-5) * g_ref[...] + be_ref[...]
  return jnp.maximum(o, 0.0)


def _fin_mid_body(layer, acc_ref, den_ref, sd_ref, lad_ref, h_ref, b_ref,
                  g_ref, be_ref, w_ref, a2_ref, hn_ref, sdn_ref):
  o = _gat_post(acc_ref, den_ref, sd_ref, lad_ref, h_ref, b_ref, g_ref,
                be_ref, layer)
  hn = jnp.dot(o, w_ref[...], preferred_element_type=_f32)
  hn_ref[...] = hn
  sdn_ref[...] = jnp.dot(hn, a2_ref[...], preferred_element_type=_f32)


def _fin_last_body(acc_ref, den_ref, sd_ref, lad_ref, h_ref, b_ref,
                   g_ref, be_ref, hn_ref):
  hn_ref[...] = _gat_post(acc_ref, den_ref, sd_ref, lad_ref, h_ref, b_ref,
                          g_ref, be_ref, 2)


_PARAM_SPECS = [
    pl.BlockSpec((NC, BLK, H), lambda i: (0, i, 0)),
    pl.BlockSpec((BLK, 1), lambda i: (i, 0)),
    pl.BlockSpec((BLK, 2), lambda i: (i, 0)),
    pl.BlockSpec((BLK, 3), lambda i: (i, 0)),
    pl.BlockSpec((BLK, H), lambda i: (i, 0)),
    pl.BlockSpec((1, H), lambda i: (0, 0)),
    pl.BlockSpec((1, H), lambda i: (0, 0)),
    pl.BlockSpec((1, H), lambda i: (0, 0)),
]


def _finalize_mid(layer, acc, den, sd, lad, h, b, g, be, w, a2):
  return pl.pallas_call(
      functools.partial(_fin_mid_body, layer),
      grid=(NBLK,),
      in_specs=_PARAM_SPECS + [
          pl.BlockSpec((H, H), lambda i: (0, 0)),
          pl.BlockSpec((H, 2), lambda i: (0, 0)),
      ],
      out_specs=[
          pl.BlockSpec((BLK, H), lambda i: (i, 0)),
          pl.BlockSpec((BLK, 2), lambda i: (i, 0)),
      ],
      out_shape=(jax.ShapeDtypeStruct((N, H), _f32),
                 jax.ShapeDtypeStruct((N, 2), _f32)),
  )(acc, den, sd, lad, h, b, g, be, w, a2)


def _finalize_last(acc, den, sd, lad, h, b, g, be):
  return pl.pallas_call(
      _fin_last_body,
      grid=(NBLK,),
      in_specs=_PARAM_SPECS,
      out_specs=pl.BlockSpec((BLK, H), lambda i: (i, 0)),
      out_shape=jax.ShapeDtypeStruct((N, H), _f32),
  )(acc, den, sd, lad, h, b, g, be)


def _head_body(h_ref, wo1_ref, bo1_ref, wo2_ref, bo2_ref, g_ref, be_ref,
               o_ref, acc_ref):
  i = pl.program_id(0)

  @pl.when(i == 0)
  def _():
    acc_ref[...] = jnp.zeros_like(acc_ref)

  acc_ref[...] += jnp.sum(h_ref[...], axis=0, keepdims=True)

  @pl.when(i == NBLK - 1)
  def _():
    hg = acc_ref[...] * (1.0 / N)
    o1 = jnp.dot(hg, wo1_ref[...], preferred_element_type=_f32) + bo1_ref[...]
    o1 = jnp.maximum(o1, 0.0)
    o2 = jnp.dot(o1, wo2_ref[...], preferred_element_type=_f32) + bo2_ref[...]
    m = jnp.mean(o2, axis=1, keepdims=True)
    v = jnp.mean((o2 - m) ** 2, axis=1, keepdims=True)
    o_ref[...] = (o2 - m) * lax.rsqrt(v + 1e-5) * g_ref[...] + be_ref[...]


def _head(h3, wo1, bo1, wo2, bo2, g, be):
  return pl.pallas_call(
      _head_body,
      grid=(NBLK,),
      in_specs=[
          pl.BlockSpec((BLK, H), lambda i: (i, 0)),
          pl.BlockSpec((H, H), lambda i: (0, 0)),
          pl.BlockSpec((1, H), lambda i: (0, 0)),
          pl.BlockSpec((H, OUT), lambda i: (0, 0)),
          pl.BlockSpec((1, OUT), lambda i: (0, 0)),
          pl.BlockSpec((1, OUT), lambda i: (0, 0)),
          pl.BlockSpec((1, OUT), lambda i: (0, 0)),
      ],
      out_specs=pl.BlockSpec((1, OUT), lambda i: (0, 0)),
      out_shape=jax.ShapeDtypeStruct((1, OUT), _f32),
      scratch_shapes=[pltpu.VMEM((1, H), _f32)],
  )(h3, wo1, bo1, wo2, bo2, g, be)


def _pad_tiles(arr):
  a = arr.reshape(NW, EPT)
  a = jnp.pad(a, ((0, 0), (0, EPT_PAD - EPT)))
  return a.reshape(NW, NCHUNK, CH)


def kernel(x, edge_index, edge_attr, W0, We0, as0, ad0, ae0, b0, lng0, lnb0,
           W1, We1, as1, ad1, ae1, b1, lng1, lnb1,
           W2, We2, as2, ad2, ae2, b2, lng2, lnb2,
           Wr1, br1, Wr2, br2, Wo1, bo1, Wo2, bo2, lngo, lnbo):
  src = edge_index[0].astype(jnp.int32)
  dst = edge_index[1].astype(jnp.int32)
  srcp = _pad_tiles(src)
  dstp = _pad_tiles(dst)

  ae3 = jnp.stack([We0 @ ae0, We1 @ ae1, We2 @ ae2], axis=1)
  m = jnp.kron(jnp.eye(32, dtype=_f32), ae3)
  ea_rs = edge_attr.reshape(N, 128)
  al_all = _alpha_e(ea_rs, m).reshape(N, 32, 3).reshape(E, 3)
  aeps = [_pad_tiles(al_all[:, l]) for l in range(3)]

  eatp = jnp.pad(edge_attr.T.reshape(4, NW, EPT),
                 ((0, 0), (0, 0), (0, EPT_PAD - EPT))).reshape(
                     4, NW, NCHUNK, CH).transpose(1, 2, 0, 3).reshape(
                     NW, NCHUNK, 4 * CH)
  degp, lap = _deg_kernel(dstp, eatp)
  sel = jnp.kron(jnp.eye(4, dtype=_f32), jnp.ones((1, NW), _f32))
  lad = _reduce_la(degp, lap.reshape(4 * NW, N), sel, ae3)

  params = [
      (W0, as0, ad0, b0, lng0, lnb0),
      (W1, as1, ad1, b1, lng1, lnb1),
      (W2, as2, ad2, b2, lng2, lnb2),
  ]
  a2s = [jnp.stack([p[1], p[2]], axis=1) for p in params]

  h, sd = _ka(x, W0, a2s[0])
  for l in range(3):
    acc, denp = _edge_kernel(h, sd[:, 0], sd[:, 1], srcp, dstp, aeps[l])
    den = _den_sum(denp)
    W, _, _, b, g, be = params[l]
    if l < 2:
      Wn = params[l + 1][0]
      h, sd = _finalize_mid(l, acc, den, sd, lad,
                            h, b.reshape(1, H), g.reshape(1, H),
                            be.reshape(1, H), Wn, a2s[l + 1])
    else:
      h3 = _finalize_last(acc, den, sd, lad, h, b.reshape(1, H),
                          g.reshape(1, H), be.reshape(1, H))

  return _head(h3, Wo1, bo1.reshape(1, H), Wo2, bo2.reshape(1, OUT),
               lngo.reshape(1, OUT), lnbo.reshape(1, OUT))


# speedup vs baseline: 13.1134x; 1.0045x over previous
import functools

import jax
import jax.numpy as jnp
from jax import lax
from jax.experimental import pallas as pl
from jax.experimental.pallas import tpu as pltpu
from jax.experimental.pallas import tpu_sc as plsc

N = 10000
E = 320000
H = 128
OUT = 256

NC = 2
NS = 16
NW = NC * NS
EPT = E // NW
CH = 64
NCHUNK = 158
EPT_PAD = NCHUNK * CH
NBODY = NCHUNK // 2
RPT = N // NS

BLK = 80
NBLK = N // BLK

_f32 = jnp.float32
_sc_mesh = plsc.VectorSubcoreMesh(
    core_axis_name="c", subcore_axis_name="s", num_cores=NC, num_subcores=NS)


def _edge_body(h_hbm, s_hbm, d_hbm, srcp_hbm, dstp_hbm, aep_hbm,
               acc_hbm, den_hbm,
               s_v, d_v, src0, src1, dst0, dst1, ae0, ae1, coef_v,
               rows0, rows1, den_v, out_sh,
               sem_i0, sem_i1, sem_g0, sem_g1, sem_s0, sem_s1):
  cid = lax.axis_index("c")
  sid = lax.axis_index("s")
  wid = cid * NS + sid

  pltpu.sync_copy(s_hbm, s_v)
  pltpu.sync_copy(d_hbm, d_v)

  zero16 = jnp.zeros((16,), _f32)
  zi16 = jnp.zeros((16,), jnp.int32)
  NG = CH // 16

  def _zden(i, carry):
    den_v[pl.ds(i * 16, 16)] = zero16
    return carry
  lax.fori_loop(0, N // 16, _zden, 0)

  def _zrows(i, carry):
    for j in range(H // 16):
      rows0[i, pl.ds(j * 16, 16)] = zero16
    return carry
  lax.fori_loop(0, CH, _zrows, 0)

  zbase = sid * 632
  for t in range(8):
    pltpu.sync_copy(rows0, out_sh.at[pl.ds(zbase + t * 64, 64)])

  @pl.when(sid < NS - 1)
  def _ztail():
    pltpu.sync_copy(rows0, out_sh.at[pl.ds(zbase + 512, 64)])
    pltpu.sync_copy(rows0.at[pl.ds(0, 56)],
                    out_sh.at[pl.ds(zbase + 576, 56)])

  @pl.when(sid == NS - 1)
  def _ztail2():
    pltpu.sync_copy(rows0.at[pl.ds(0, 8)],
                    out_sh.at[pl.ds(zbase + 512, 8)])
  plsc.subcore_barrier()

  def _issue_idx(c, srcb, dstb, aeb, semi):
    pltpu.async_copy(srcp_hbm.at[wid, c], srcb, semi)
    pltpu.async_copy(dstp_hbm.at[wid, c], dstb, semi)
    pltpu.async_copy(aep_hbm.at[wid, c], aeb, semi)

  def _wait_idx(c, srcb, dstb, aeb, semi):
    pltpu.make_async_copy(srcp_hbm.at[wid, c], srcb, semi).wait()
    pltpu.make_async_copy(dstp_hbm.at[wid, c], dstb, semi).wait()
    pltpu.make_async_copy(aep_hbm.at[wid, c], aeb, semi).wait()

  def _issue_gather(srcb, rowsb, semg):
    pltpu.async_copy(h_hbm.at[srcb], rowsb, semg)

  def _wait_gather(srcb, rowsb, semg):
    pltpu.make_async_copy(h_hbm.at[srcb], rowsb, semg).wait()

  def _issue_scatter(dstb, rowsb, sems):
    pltpu.async_copy(rowsb, out_sh.at[dstb], sems, add=True)

  def _wait_scatter(dstb, rowsb, sems):
    pltpu.make_async_copy(rowsb, out_sh.at[dstb], sems).wait()

  def _process(c, srcb, dstb, aeb, rowsb):
    for g in range(NG):
      si = srcb[pl.ds(g * 16, 16)]
      di = dstb[pl.ds(g * 16, 16)]
      ae = aeb[pl.ds(g * 16, 16)]
      sv = plsc.load_gather(s_v, [si])
      dv = plsc.load_gather(d_v, [di])
      al = sv + dv + ae
      al = jnp.maximum(al, al * 0.2)
      ex = jnp.exp(al)
      valid = (c * CH + g * 16) < EPT
      ex = jnp.where(valid, ex, zero16)
      coef_v[pl.ds(g * 16, 16)] = ex
      plsc.addupdate_scatter(den_v, [di], ex)
    def _scale(rb, carry2):
      base = rb * 8
      for k in range(8):
        r = base + k
        cv = plsc.load_gather(coef_v, [jnp.full((16,), r, jnp.int32)])
        for j in range(H // 16):
          rowsb[r, pl.ds(j * 16, 16)] = rowsb[r, pl.ds(j * 16, 16)] * cv
      return carry2
    lax.fori_loop(0, CH // 8, _scale, 0)

  _issue_idx(0, src0, dst0, ae0, sem_i0)
  _issue_idx(1, src1, dst1, ae1, sem_i1)
  _wait_idx(0, src0, dst0, ae0, sem_i0)
  _issue_gather(src0, rows0, sem_g0)

  def _body(i, carry):
    c0 = 2 * i
    c1 = 2 * i + 1
    _wait_gather(src0, rows0, sem_g0)

    @pl.when(i > 0)
    def _():
      _wait_scatter(dst1, rows1, sem_s1)
      _issue_idx(c1, src1, dst1, ae1, sem_i1)

    _wait_idx(c1, src1, dst1, ae1, sem_i1)
    _issue_gather(src1, rows1, sem_g1)
    _process(c0, src0, dst0, ae0, rows0)
    _issue_scatter(dst0, rows0, sem_s0)

    _wait_gather(src1, rows1, sem_g1)
    _process(c1, src1, dst1, ae1, rows1)
    _issue_scatter(dst1, rows1, sem_s1)

    @pl.when(i < NBODY - 1)
    def _():
      _wait_scatter(dst0, rows0, sem_s0)
      _issue_idx(c0 + 2, src0, dst0, ae0, sem_i0)
      _wait_idx(c0 + 2, src0, dst0, ae0, sem_i0)
      _issue_gather(src0, rows0, sem_g0)
    return carry
  lax.fori_loop(0, NBODY, _body, 0)

  _wait_scatter(dst0, rows0, sem_s0)
  _wait_scatter(dst1, rows1, sem_s1)

  plsc.subcore_barrier()
  pltpu.sync_copy(den_v, den_hbm.at[wid])
  for t in range(8):
    pltpu.sync_copy(out_sh.at[pl.ds(zbase + t * 64, 64)],
                    acc_hbm.at[cid, pl.ds(zbase + t * 64, 64)])

  @pl.when(sid < NS - 1)
  def _dtail():
    pltpu.sync_copy(out_sh.at[pl.ds(zbase + 512, 120)],
                    acc_hbm.at[cid, pl.ds(zbase + 512, 120)])

  @pl.when(sid == NS - 1)
  def _dtail2():
    pltpu.sync_copy(out_sh.at[pl.ds(zbase + 512, 8)],
                    acc_hbm.at[cid, pl.ds(zbase + 512, 8)])


_edge_kernel = functools.partial(
    pl.kernel,
    out_type=(jax.ShapeDtypeStruct((NC, N, H), _f32),
              jax.ShapeDtypeStruct((NW, N), _f32)),
    mesh=_sc_mesh,
    compiler_params=pltpu.CompilerParams(needs_layout_passes=False),
    scratch_types=[
        pltpu.VMEM((N,), _f32),
        pltpu.VMEM((N,), _f32),
        pltpu.VMEM((CH,), jnp.int32),
        pltpu.VMEM((CH,), jnp.int32),
        pltpu.VMEM((CH,), jnp.int32),
        pltpu.VMEM((CH,), jnp.int32),
        pltpu.VMEM((CH,), _f32),
        pltpu.VMEM((CH,), _f32),
        pltpu.VMEM((CH,), _f32),
        pltpu.VMEM((CH, H), _f32),
        pltpu.VMEM((CH, H), _f32),
        pltpu.VMEM((N,), _f32),
        pltpu.VMEM_SHARED((N, H), _f32),
        pltpu.SemaphoreType.DMA,
        pltpu.SemaphoreType.DMA,
        pltpu.SemaphoreType.DMA,
        pltpu.SemaphoreType.DMA,
        pltpu.SemaphoreType.DMA,
        pltpu.SemaphoreType.DMA,
    ],
)(_edge_body)


def _deg_body(dstp_hbm, eatp_hbm, degp_hbm, lap_hbm,
              dst_c, eat_c, deg_v, la0, la1, la2, la3):
  cid = lax.axis_index("c")
  sid = lax.axis_index("s")
  wid = cid * NS + sid
  las = (la0, la1, la2, la3)

  zero16 = jnp.zeros((16,), _f32)
  one16 = jnp.ones((16,), _f32)

  def _z(i, carry):
    deg_v[pl.ds(i * 16, 16)] = zero16
    for j in range(4):
      las[j][pl.ds(i * 16, 16)] = zero16
    return carry
  lax.fori_loop(0, N // 16, _z, 0)

  def _chunk(c, carry):
    pltpu.sync_copy(dstp_hbm.at[wid, c], dst_c)
    pltpu.sync_copy(eatp_hbm.at[wid, c], eat_c)
    for g in range(CH // 16):
      di = dst_c[pl.ds(g * 16, 16)]
      valid = (c * CH + g * 16) < EPT
      ones_m = jnp.where(valid, one16, zero16)
      plsc.addupdate_scatter(deg_v, [di], ones_m)
      for j in range(4):
        ev = eat_c[pl.ds(j * CH + g * 16, 16)]
        ev = jnp.where(valid, ev, zero16)
        plsc.addupdate_scatter(las[j], [di], ev)
    return carry
  lax.fori_loop(0, NCHUNK, _chunk, 0)

  pltpu.sync_copy(deg_v, degp_hbm.at[wid])
  for j in range(4):
    pltpu.sync_copy(las[j], lap_hbm.at[j, wid])


_deg_kernel = functools.partial(
    pl.kernel,
    out_type=(jax.ShapeDtypeStruct((NW, N), _f32),
              jax.ShapeDtypeStruct((4, NW, N), _f32)),
    mesh=_sc_mesh,
    compiler_params=pltpu.CompilerParams(needs_layout_passes=False),
    scratch_types=[
        pltpu.VMEM((CH,), jnp.int32),
        pltpu.VMEM((4 * CH,), _f32),
        pltpu.VMEM((N,), _f32),
        pltpu.VMEM((N,), _f32),
        pltpu.VMEM((N,), _f32),
        pltpu.VMEM((N,), _f32),
        pltpu.VMEM((N,), _f32),
    ],
)(_deg_body)


def _ka_body(x_ref, w_ref, a2_ref, h_ref, sd_ref):
  h = jnp.dot(x_ref[...], w_ref[...], preferred_element_type=_f32)
  h_ref[...] = h
  sd_ref[...] = jnp.dot(h, a2_ref[...], preferred_element_type=_f32)


def _ka(x, w, a2):
  return pl.pallas_call(
      _ka_body,
      grid=(NBLK,),
      in_specs=[
          pl.BlockSpec((BLK, x.shape[1]), lambda i: (i, 0)),
          pl.BlockSpec(w.shape, lambda i: (0, 0)),
          pl.BlockSpec(a2.shape, lambda i: (0, 0)),
      ],
      out_specs=[
          pl.BlockSpec((BLK, H), lambda i: (i, 0)),
          pl.BlockSpec((BLK, 2), lambda i: (i, 0)),
      ],
      out_shape=(jax.ShapeDtypeStruct((N, H), _f32),
                 jax.ShapeDtypeStruct((N, 2), _f32)),
  )(x, w, a2)


def _mm_body(x_ref, m_ref, o_ref):
  o_ref[...] = jnp.dot(x_ref[...], m_ref[...], preferred_element_type=_f32)


def _alpha_e(ea_rs, m):
  return pl.pallas_call(
      _mm_body,
      grid=(NBLK,),
      in_specs=[
          pl.BlockSpec((BLK, 128), lambda i: (i, 0)),
          pl.BlockSpec(m.shape, lambda i: (0, 0)),
      ],
      out_specs=pl.BlockSpec((BLK, m.shape[1]), lambda i: (i, 0)),
      out_shape=jax.ShapeDtypeStruct((N, m.shape[1]), _f32),
  )(ea_rs, m)


def _red_body(degp_ref, lap_ref, sel_ref, ae3_ref, out_ref):
  deg = jnp.maximum(jnp.sum(degp_ref[...], axis=0), 1.0)
  la4 = lax.dot_general(sel_ref[...], lap_ref[...],
                        (((1,), (0,)), ((), ())),
                        preferred_element_type=_f32)
  la4 = la4 / deg[None, :]
  out_ref[...] = lax.dot_general(la4, ae3_ref[...],
                                 (((0,), (0,)), ((), ())),
                                 preferred_element_type=_f32)


def _reduce_la(degp, lap128, sel, ae3):
  return pl.pallas_call(
      _red_body,
      out_shape=jax.ShapeDtypeStruct((N, 3), _f32),
  )(degp, lap128, sel, ae3)


def _densum_body(denp_ref, out_ref):
  out_ref[...] = lax.dot_general(denp_ref[...], jnp.ones((NW, 1), _f32),
                                 (((0,), (0,)), ((), ())),
                                 preferred_element_type=_f32)


def _den_sum(denp):
  return pl.pallas_call(
      _densum_body,
      out_shape=jax.ShapeDtypeStruct((N, 1), _f32),
  )(denp)


def _gat_post(acc_ref, den_ref, sd_ref, lad_ref, h_ref, b_ref, g_ref,
              be_ref, layer):
  s = sd_ref[:, 0]
  dd = sd_ref[:, 1]
  asl = s + dd + lad_ref[:, layer]
  asl = jnp.maximum(asl, asl * 0.2)
  exs = jnp.exp(asl)
  h = h_ref[...]
  num = acc_ref[0] + acc_ref[1] + exs[:, None] * h
  den = den_ref[...] + exs[:, None]
  o = num / den + b_ref[...]
  m = jnp.mean(o, axis=1, keepdims=True)
  v = jnp.mean((o - m) ** 2, axis=1, keepdims=True)
  o = (o - m) * lax.rsqrt(v + 1e-5) * g_ref[...] + be_ref[...]
  return jnp.maximum(o, 0.0)


def _fin_mid_body(layer, acc_ref, den_ref, sd_ref, lad_ref, h_ref, b_ref,
                  g_ref, be_ref, w_ref, a2_ref, hn_ref, sdn_ref):
  o = _gat_post(acc_ref, den_ref, sd_ref, lad_ref, h_ref, b_ref, g_ref,
                be_ref, layer)
  hn = jnp.dot(o, w_ref[...], preferred_element_type=_f32)
  hn_ref[...] = hn
  sdn_ref[...] = jnp.dot(hn, a2_ref[...], preferred_element_type=_f32)


def _fin_last_body(acc_ref, den_ref, sd_ref, lad_ref, h_ref, b_ref,
                   g_ref, be_ref, hn_ref):
  hn_ref[...] = _gat_post(acc_ref, den_ref, sd_ref, lad_ref, h_ref, b_ref,
                          g_ref, be_ref, 2)


_PARAM_SPECS = [
    pl.BlockSpec((NC, BLK, H), lambda i: (0, i, 0)),
    pl.BlockSpec((BLK, 1), lambda i: (i, 0)),
    pl.BlockSpec((BLK, 2), lambda i: (i, 0)),
    pl.BlockSpec((BLK, 3), lambda i: (i, 0)),
    pl.BlockSpec((BLK, H), lambda i: (i, 0)),
    pl.BlockSpec((1, H), lambda i: (0, 0)),
    pl.BlockSpec((1, H), lambda i: (0, 0)),
    pl.BlockSpec((1, H), lambda i: (0, 0)),
]


def _finalize_mid(layer, acc, den, sd, lad, h, b, g, be, w, a2):
  return pl.pallas_call(
      functools.partial(_fin_mid_body, layer),
      grid=(NBLK,),
      in_specs=_PARAM_SPECS + [
          pl.BlockSpec((H, H), lambda i: (0, 0)),
          pl.BlockSpec((H, 2), lambda i: (0, 0)),
      ],
      out_specs=[
          pl.BlockSpec((BLK, H), lambda i: (i, 0)),
          pl.BlockSpec((BLK, 2), lambda i: (i, 0)),
      ],
      out_shape=(jax.ShapeDtypeStruct((N, H), _f32),
                 jax.ShapeDtypeStruct((N, 2), _f32)),
  )(acc, den, sd, lad, h, b, g, be, w, a2)


def _finalize_last(acc, den, sd, lad, h, b, g, be):
  return pl.pallas_call(
      _fin_last_body,
      grid=(NBLK,),
      in_specs=_PARAM_SPECS,
      out_specs=pl.BlockSpec((BLK, H), lambda i: (i, 0)),
      out_shape=jax.ShapeDtypeStruct((N, H), _f32),
  )(acc, den, sd, lad, h, b, g, be)


def _head_body(h_ref, wo1_ref, bo1_ref, wo2_ref, bo2_ref, g_ref, be_ref,
               o_ref, acc_ref):
  i = pl.program_id(0)

  @pl.when(i == 0)
  def _():
    acc_ref[...] = jnp.zeros_like(acc_ref)

  acc_ref[...] += jnp.sum(h_ref[...], axis=0, keepdims=True)

  @pl.when(i == NBLK - 1)
  def _():
    hg = acc_ref[...] * (1.0 / N)
    o1 = jnp.dot(hg, wo1_ref[...], preferred_element_type=_f32) + bo1_ref[...]
    o1 = jnp.maximum(o1, 0.0)
    o2 = jnp.dot(o1, wo2_ref[...], preferred_element_type=_f32) + bo2_ref[...]
    m = jnp.mean(o2, axis=1, keepdims=True)
    v = jnp.mean((o2 - m) ** 2, axis=1, keepdims=True)
    o_ref[...] = (o2 - m) * lax.rsqrt(v + 1e-5) * g_ref[...] + be_ref[...]


def _head(h3, wo1, bo1, wo2, bo2, g, be):
  return pl.pallas_call(
      _head_body,
      grid=(NBLK,),
      in_specs=[
          pl.BlockSpec((BLK, H), lambda i: (i, 0)),
          pl.BlockSpec((H, H), lambda i: (0, 0)),
          pl.BlockSpec((1, H), lambda i: (0, 0)),
          pl.BlockSpec((H, OUT), lambda i: (0, 0)),
          pl.BlockSpec((1, OUT), lambda i: (0, 0)),
          pl.BlockSpec((1, OUT), lambda i: (0, 0)),
          pl.BlockSpec((1, OUT), lambda i: (0, 0)),
      ],
      out_specs=pl.BlockSpec((1, OUT), lambda i: (0, 0)),
      out_shape=jax.ShapeDtypeStruct((1, OUT), _f32),
      scratch_shapes=[pltpu.VMEM((1, H), _f32)],
  )(h3, wo1, bo1, wo2, bo2, g, be)


def _pad_tiles(arr):
  a = arr.reshape(NW, EPT)
  a = jnp.pad(a, ((0, 0), (0, EPT_PAD - EPT)))
  return a.reshape(NW, NCHUNK, CH)


def kernel(x, edge_index, edge_attr, W0, We0, as0, ad0, ae0, b0, lng0, lnb0,
           W1, We1, as1, ad1, ae1, b1, lng1, lnb1,
           W2, We2, as2, ad2, ae2, b2, lng2, lnb2,
           Wr1, br1, Wr2, br2, Wo1, bo1, Wo2, bo2, lngo, lnbo):
  src = edge_index[0].astype(jnp.int32)
  dst = edge_index[1].astype(jnp.int32)
  srcp = _pad_tiles(src)
  dstp = _pad_tiles(dst)

  ae3 = jnp.stack([We0 @ ae0, We1 @ ae1, We2 @ ae2], axis=1)
  m = jnp.kron(jnp.eye(32, dtype=_f32), ae3)
  ea_rs = edge_attr.reshape(N, 128)
  al_all = _alpha_e(ea_rs, m).reshape(N, 32, 3).reshape(E, 3)
  aeps = [_pad_tiles(al_all[:, l]) for l in range(3)]

  eatp = jnp.pad(edge_attr.T.reshape(4, NW, EPT),
                 ((0, 0), (0, 0), (0, EPT_PAD - EPT))).reshape(
                     4, NW, NCHUNK, CH).transpose(1, 2, 0, 3).reshape(
                     NW, NCHUNK, 4 * CH)
  degp, lap = _deg_kernel(dstp, eatp)
  sel = jnp.kron(jnp.eye(4, dtype=_f32), jnp.ones((1, NW), _f32))
  lad = _reduce_la(degp, lap.reshape(4 * NW, N), sel, ae3)

  params = [
      (W0, as0, ad0, b0, lng0, lnb0),
      (W1, as1, ad1, b1, lng1, lnb1),
      (W2, as2, ad2, b2, lng2, lnb2),
  ]
  a2s = [jnp.stack([p[1], p[2]], axis=1) for p in params]

  h, sd = _ka(x, W0, a2s[0])
  for l in range(3):
    acc, denp = _edge_kernel(h, sd[:, 0], sd[:, 1], srcp, dstp, aeps[l])
    den = _den_sum(denp)
    W, _, _, b, g, be = params[l]
    if l < 2:
      Wn = params[l + 1][0]
      h, sd = _finalize_mid(l, acc, den, sd, lad,
                            h, b.reshape(1, H), g.reshape(1, H),
                            be.reshape(1, H), Wn, a2s[l + 1])
    else:
      h3 = _finalize_last(acc, den, sd, lad, h, b.reshape(1, H),
                          g.reshape(1, H), be.reshape(1, H))

  return _head(h3, Wo1, bo1.reshape(1, H), Wo2, bo2.reshape(1, OUT),
               lngo.reshape(1, OUT), lnbo.reshape(1, OUT))
